# baseline (speedup 1.0000x reference)
import functools

import jax
import jax.numpy as jnp
from jax import lax
from jax.experimental import pallas as pl
from jax.experimental.pallas import tpu as pltpu
from jax.experimental.pallas import tpu_sc as plsc

_GROUP = 12500
_B = 1000
_NCHUNK = 100
_NW = 32
_SUB = 200
_NSUB = _B // _SUB
_EPS = 1e-6


def _sc_stats(h_ref, out_ref, buf0, buf1, acc_ref, sem):
    c = lax.axis_index("c")
    s = lax.axis_index("s")
    wid = s * 2 + c
    extra = _NCHUNK - 3 * _NW
    nmine = jnp.where(wid < extra, 4, 3)
    bufs = (buf0, buf1)

    for g in range(16 * 16):
        acc_ref[0, pl.ds(g * 16, 16)] = jnp.zeros((16,), jnp.float32)

    def do_chunk(k, _):
        ch = wid + k * _NW
        r0 = ch * _B
        pltpu.make_async_copy(
            h_ref.at[pl.ds(r0, _SUB), :], bufs[0], sem).start()
        for sub in range(_NSUB):
            buf = bufs[sub % 2]
            pltpu.make_async_copy(
                h_ref.at[pl.ds(r0 + sub * _SUB, _SUB), :], buf, sem).wait()
            if sub < _NSUB - 1:
                pltpu.make_async_copy(
                    h_ref.at[pl.ds(r0 + (sub + 1) * _SUB, _SUB), :],
                    bufs[(sub + 1) % 2], sem).start()

            gr0 = r0 + sub * _SUB

            def do_row(r, _, buf=buf, gr0=gr0):
                off = lax.div(gr0 + r, _GROUP) * 512
                for g in range(16):
                    x = buf[r, pl.ds(g * 16, 16)]
                    plsc.addupdate(acc_ref.at[0, pl.ds(off + g * 16, 16)], x)
                    plsc.addupdate(
                        acc_ref.at[0, pl.ds(off + 256 + g * 16, 16)], x * x)
                return 0

            lax.fori_loop(0, _SUB, do_row, 0)
        return 0

    lax.fori_loop(0, nmine, do_chunk, 0)
    pltpu.sync_copy(acc_ref, out_ref.at[wid])


def _tc_norm(h_ref, p_ref, w_ref, b_ref, ms_ref, o_ref, coef_ref):
    t = pl.program_id(0)
    inv_n = 1.0 / _GROUP

    @pl.when(t == 0)
    def _coefs():
        for j in range(8):
            sm = jnp.sum(p_ref[:, 0, 512 * j:512 * j + 256], axis=0,
                         keepdims=True)
            ss = jnp.sum(p_ref[:, 0, 512 * j + 256:512 * j + 512], axis=0,
                         keepdims=True)
            m = sm * inv_n
            mm = m * ms_ref[...]
            var = ss * inv_n - (2.0 * m - mm) * mm
            a = w_ref[...] * jax.lax.rsqrt(var + _EPS)
            coef_ref[2 * j:2 * j + 1, :] = a
            coef_ref[2 * j + 1:2 * j + 2, :] = b_ref[...] - a * mm

    pos = jax.lax.rem(t * _B, _GROUP)
    seg = jax.lax.div(t * _B, _GROUP)
    straddle = pos + _B > _GROUP
    y = h_ref[...]
    a0 = coef_ref[pl.ds(2 * seg, 1), :]
    c0 = coef_ref[pl.ds(2 * seg + 1, 1), :]

    @pl.when(jnp.logical_not(straddle))
    def _():
        o_ref[...] = y * a0 + c0

    @pl.when(straddle)
    def _():
        split = _GROUP - (_GROUP // _B) * _B
        rowid = jax.lax.broadcasted_iota(jnp.int32, (_B, 256), 0)
        a1 = coef_ref[pl.ds(2 * seg + 2, 1), :]
        c1 = coef_ref[pl.ds(2 * seg + 3, 1), :]
        o_ref[...] = jnp.where(rowid < split, y * a0 + c0, y * a1 + c1)


def kernel(h, weight, bias, mean_scale):
    n_rows, d = h.shape
    hf = h.astype(jnp.float32)
    w2 = weight.astype(jnp.float32).reshape(1, d)
    b2 = bias.astype(jnp.float32).reshape(1, d)
    ms2 = mean_scale.astype(jnp.float32).reshape(1, d)

    mesh = plsc.VectorSubcoreMesh(core_axis_name="c", subcore_axis_name="s")
    sc_fn = functools.partial(
        pl.kernel,
        mesh=mesh,
        out_type=jax.ShapeDtypeStruct((_NW, 1, 4096), jnp.float32),
        scratch_types=[
            pltpu.VMEM((_SUB, d), jnp.float32),
            pltpu.VMEM((_SUB, d), jnp.float32),
            pltpu.VMEM((1, 4096), jnp.float32),
            pltpu.SemaphoreType.DMA,
        ],
    )(_sc_stats)
    partials = sc_fn(hf)

    out = pl.pallas_call(
        _tc_norm,
        grid=(n_rows // _B,),
        in_specs=[
            pl.BlockSpec((_B, d), lambda t: (t, 0)),
            pl.BlockSpec((_NW, 1, 4096), lambda t: (0, 0, 0)),
            pl.BlockSpec((1, d), lambda t: (0, 0)),
            pl.BlockSpec((1, d), lambda t: (0, 0)),
            pl.BlockSpec((1, d), lambda t: (0, 0)),
        ],
        out_specs=pl.BlockSpec((_B, d), lambda t: (t, 0)),
        out_shape=jax.ShapeDtypeStruct((n_rows, d), jnp.float32),
        scratch_shapes=[pltpu.VMEM((16, 256), jnp.float32)],
    )(hf, partials, w2, b2, ms2)

    return out.astype(h.dtype)


# speedup vs baseline: 3.6856x; 3.6856x over previous
import jax
import jax.numpy as jnp
from jax.experimental import pallas as pl
from jax.experimental.pallas import tpu as pltpu

_GROUP = 12500
_B = 1000
_LAG = 13
_RING = 16
_EPS = 1e-6


def _gn_kernel(h_ref, w_ref, b_ref, ms_ref, o_ref, slab_ref, sums_ref, coef_ref):
    s = pl.program_id(0)
    n_in = pl.num_programs(0) - _LAG
    inv_n = 1.0 / _GROUP

    @pl.when(s < n_in)
    def _ingest():
        x = h_ref[...]
        slot = jax.lax.rem(s, _RING)
        slab_ref[pl.ds(slot * _B, _B), :] = x
        pos = jax.lax.rem(s * _B, _GROUP)
        seg = jax.lax.div(s * _B, _GROUP)

        def psums(xm):
            return (jnp.sum(xm, axis=0, keepdims=True),
                    jnp.sum(xm * xm, axis=0, keepdims=True))

        @pl.when(pos == 0)
        def _():
            ps, pss = psums(x)
            sums_ref[pl.ds(2 * seg, 1), :] = ps
            sums_ref[pl.ds(2 * seg + 1, 1), :] = pss

        @pl.when((pos > 0) & (pos + _B <= _GROUP))
        def _():
            ps, pss = psums(x)
            sums_ref[pl.ds(2 * seg, 1), :] += ps
            sums_ref[pl.ds(2 * seg + 1, 1), :] += pss

        @pl.when(pos + _B > _GROUP)
        def _():
            split = _GROUP - (_GROUP // _B) * _B
            ps, pss = psums(x[:split])
            sums_ref[pl.ds(2 * seg, 1), :] += ps
            sums_ref[pl.ds(2 * seg + 1, 1), :] += pss
            ps2, pss2 = psums(x[split:])
            sums_ref[pl.ds(2 * seg + 2, 1), :] = ps2
            sums_ref[pl.ds(2 * seg + 3, 1), :] = pss2

    @pl.when(s >= _LAG)
    def _emit():
        e = s - _LAG
        pos = jax.lax.rem(e * _B, _GROUP)
        seg = jax.lax.div(e * _B, _GROUP)
        straddle = pos + _B > _GROUP

        def finalize(j):
            sm = sums_ref[pl.ds(2 * j, 1), :]
            ss = sums_ref[pl.ds(2 * j + 1, 1), :]
            m = sm * inv_n
            mm = m * ms_ref[...]
            var = ss * inv_n - (2.0 * m - mm) * mm
            a = w_ref[...] * jax.lax.rsqrt(var + _EPS)
            coef_ref[pl.ds(2 * j, 1), :] = a
            coef_ref[pl.ds(2 * j + 1, 1), :] = b_ref[...] - a * mm

        @pl.when(pos == 0)
        def _():
            finalize(seg)

        @pl.when(straddle)
        def _():
            finalize(seg + 1)

        slot = jax.lax.rem(e, _RING)
        y = slab_ref[pl.ds(slot * _B, _B), :]
        a0 = coef_ref[pl.ds(2 * seg, 1), :]
        c0 = coef_ref[pl.ds(2 * seg + 1, 1), :]

        @pl.when(jnp.logical_not(straddle))
        def _():
            o_ref[...] = y * a0 + c0

        @pl.when(straddle)
        def _():
            split = _GROUP - (_GROUP // _B) * _B
            rowid = jax.lax.broadcasted_iota(jnp.int32, (_B, 256), 0)
            a1 = coef_ref[pl.ds(2 * seg + 2, 1), :]
            c1 = coef_ref[pl.ds(2 * seg + 3, 1), :]
            o_ref[...] = jnp.where(rowid < split, y * a0 + c0, y * a1 + c1)


def kernel(h, weight, bias, mean_scale):
    n_rows, d = h.shape
    n_blk = n_rows // _B
    hf = h.astype(jnp.float32)
    w2 = weight.astype(jnp.float32).reshape(1, d)
    b2 = bias.astype(jnp.float32).reshape(1, d)
    ms2 = mean_scale.astype(jnp.float32).reshape(1, d)

    out = pl.pallas_call(
        _gn_kernel,
        grid=(n_blk + _LAG,),
        in_specs=[
            pl.BlockSpec((_B, d), lambda s: (jnp.minimum(s, n_blk - 1), 0)),
            pl.BlockSpec((1, d), lambda s: (0, 0)),
            pl.BlockSpec((1, d), lambda s: (0, 0)),
            pl.BlockSpec((1, d), lambda s: (0, 0)),
        ],
        out_specs=pl.BlockSpec(
            (_B, d), lambda s: (jnp.maximum(s - _LAG, 0), 0)
        ),
        out_shape=jax.ShapeDtypeStruct((n_rows, d), jnp.float32),
        scratch_shapes=[
            pltpu.VMEM((_RING * _B, 256), jnp.float32),
            pltpu.VMEM((16, 256), jnp.float32),
            pltpu.VMEM((16, 256), jnp.float32),
        ],
    )(hf, w2, b2, ms2)

    return out.astype(h.dtype)


# speedup vs baseline: 4.7802x; 1.2970x over previous
import jax
import jax.numpy as jnp
from jax.experimental import pallas as pl
from jax.experimental.pallas import tpu as pltpu

_GROUP = 12500
_B = 2000
_LAG = 7
_RING = 8
_EPS = 1e-6


def _gn_kernel(h_ref, w_ref, b_ref, ms_ref, o_ref, slab_ref, sums_ref, coef_ref):
    s = pl.program_id(0)
    n_in = pl.num_programs(0) - _LAG
    inv_n = 1.0 / _GROUP

    @pl.when(s < n_in)
    def _ingest():
        x = h_ref[...]
        slot = jax.lax.rem(s, _RING)
        slab_ref[pl.ds(slot * _B, _B), :] = x
        pos = jax.lax.rem(s * _B, _GROUP)
        seg = jax.lax.div(s * _B, _GROUP)
        split = _GROUP - pos

        def psums(xm):
            return (jnp.sum(xm, axis=0, keepdims=True),
                    jnp.sum(xm * xm, axis=0, keepdims=True))

        @pl.when(pos == 0)
        def _():
            ps, pss = psums(x)
            sums_ref[pl.ds(2 * seg, 1), :] = ps
            sums_ref[pl.ds(2 * seg + 1, 1), :] = pss

        @pl.when((pos > 0) & (pos + _B <= _GROUP))
        def _():
            ps, pss = psums(x)
            sums_ref[pl.ds(2 * seg, 1), :] += ps
            sums_ref[pl.ds(2 * seg + 1, 1), :] += pss

        @pl.when(pos + _B > _GROUP)
        def _():
            rowid = jax.lax.broadcasted_iota(jnp.int32, (_B, 256), 0)
            lo = rowid < split
            ps, pss = psums(jnp.where(lo, x, 0.0))
            sums_ref[pl.ds(2 * seg, 1), :] += ps
            sums_ref[pl.ds(2 * seg + 1, 1), :] += pss
            ps2, pss2 = psums(jnp.where(lo, 0.0, x))
            sums_ref[pl.ds(2 * seg + 2, 1), :] = ps2
            sums_ref[pl.ds(2 * seg + 3, 1), :] = pss2

    @pl.when(s >= _LAG)
    def _emit():
        e = s - _LAG
        pos = jax.lax.rem(e * _B, _GROUP)
        seg = jax.lax.div(e * _B, _GROUP)
        split = _GROUP - pos
        straddle = pos + _B > _GROUP

        def finalize(j):
            sm = sums_ref[pl.ds(2 * j, 1), :]
            ss = sums_ref[pl.ds(2 * j + 1, 1), :]
            m = sm * inv_n
            mm = m * ms_ref[...]
            var = ss * inv_n - (2.0 * m - mm) * mm
            a = w_ref[...] * jax.lax.rsqrt(var + _EPS)
            coef_ref[pl.ds(2 * j, 1), :] = a
            coef_ref[pl.ds(2 * j + 1, 1), :] = b_ref[...] - a * mm

        @pl.when(pos == 0)
        def _():
            finalize(seg)

        @pl.when(straddle)
        def _():
            finalize(seg + 1)

        slot = jax.lax.rem(e, _RING)
        y = slab_ref[pl.ds(slot * _B, _B), :]
        a0 = coef_ref[pl.ds(2 * seg, 1), :]
        c0 = coef_ref[pl.ds(2 * seg + 1, 1), :]

        @pl.when(jnp.logical_not(straddle))
        def _():
            o_ref[...] = y * a0 + c0

        @pl.when(straddle)
        def _():
            rowid = jax.lax.broadcasted_iota(jnp.int32, (_B, 256), 0)
            a1 = coef_ref[pl.ds(2 * seg + 2, 1), :]
            c1 = coef_ref[pl.ds(2 * seg + 3, 1), :]
            o_ref[...] = jnp.where(rowid < split, y * a0 + c0, y * a1 + c1)


def kernel(h, weight, bias, mean_scale):
    n_rows, d = h.shape
    n_blk = n_rows // _B
    hf = h.astype(jnp.float32)
    w2 = weight.astype(jnp.float32).reshape(1, d)
    b2 = bias.astype(jnp.float32).reshape(1, d)
    ms2 = mean_scale.astype(jnp.float32).reshape(1, d)

    out = pl.pallas_call(
        _gn_kernel,
        grid=(n_blk + _LAG,),
        in_specs=[
            pl.BlockSpec((_B, d), lambda s: (jnp.minimum(s, n_blk - 1), 0)),
            pl.BlockSpec((1, d), lambda s: (0, 0)),
            pl.BlockSpec((1, d), lambda s: (0, 0)),
            pl.BlockSpec((1, d), lambda s: (0, 0)),
        ],
        out_specs=pl.BlockSpec(
            (_B, d), lambda s: (jnp.maximum(s - _LAG, 0), 0)
        ),
        out_shape=jax.ShapeDtypeStruct((n_rows, d), jnp.float32),
        scratch_shapes=[
            pltpu.VMEM((_RING * _B, 256), jnp.float32),
            pltpu.VMEM((16, 256), jnp.float32),
            pltpu.VMEM((16, 256), jnp.float32),
        ],
    )(hf, w2, b2, ms2)

    return out.astype(h.dtype)


# speedup vs baseline: 5.5153x; 1.1538x over previous
import jax
import jax.numpy as jnp
from jax.experimental import pallas as pl
from jax.experimental.pallas import tpu as pltpu

_GROUP = 12500
_B = 4000
_LAG = 4
_RING = 5
_EPS = 1e-6


def _gn_kernel(h_ref, w_ref, b_ref, ms_ref, o_ref, slab_ref, sums_ref, coef_ref):
    s = pl.program_id(0)
    n_in = pl.num_programs(0) - _LAG
    inv_n = 1.0 / _GROUP

    @pl.when(s < n_in)
    def _ingest():
        x = h_ref[...]
        slot = jax.lax.rem(s, _RING)
        slab_ref[pl.ds(slot * _B, _B), :] = x
        pos = jax.lax.rem(s * _B, _GROUP)
        seg = jax.lax.div(s * _B, _GROUP)
        split = _GROUP - pos

        def psums(xm):
            return (jnp.sum(xm, axis=0, keepdims=True),
                    jnp.sum(xm * xm, axis=0, keepdims=True))

        @pl.when(pos == 0)
        def _():
            ps, pss = psums(x)
            sums_ref[pl.ds(2 * seg, 1), :] = ps
            sums_ref[pl.ds(2 * seg + 1, 1), :] = pss

        @pl.when((pos > 0) & (pos + _B <= _GROUP))
        def _():
            ps, pss = psums(x)
            sums_ref[pl.ds(2 * seg, 1), :] += ps
            sums_ref[pl.ds(2 * seg + 1, 1), :] += pss

        @pl.when(pos + _B > _GROUP)
        def _():
            rowid = jax.lax.broadcasted_iota(jnp.int32, (_B, 256), 0)
            lo = rowid < split
            ps, pss = psums(jnp.where(lo, x, 0.0))
            sums_ref[pl.ds(2 * seg, 1), :] += ps
            sums_ref[pl.ds(2 * seg + 1, 1), :] += pss
            ps2, pss2 = psums(jnp.where(lo, 0.0, x))
            sums_ref[pl.ds(2 * seg + 2, 1), :] = ps2
            sums_ref[pl.ds(2 * seg + 3, 1), :] = pss2

    @pl.when(s >= _LAG)
    def _emit():
        e = s - _LAG
        pos = jax.lax.rem(e * _B, _GROUP)
        seg = jax.lax.div(e * _B, _GROUP)
        split = _GROUP - pos
        straddle = pos + _B > _GROUP

        def finalize(j):
            sm = sums_ref[pl.ds(2 * j, 1), :]
            ss = sums_ref[pl.ds(2 * j + 1, 1), :]
            m = sm * inv_n
            mm = m * ms_ref[...]
            var = ss * inv_n - (2.0 * m - mm) * mm
            a = w_ref[...] * jax.lax.rsqrt(var + _EPS)
            coef_ref[pl.ds(2 * j, 1), :] = a
            coef_ref[pl.ds(2 * j + 1, 1), :] = b_ref[...] - a * mm

        @pl.when(pos == 0)
        def _():
            finalize(seg)

        @pl.when(straddle)
        def _():
            finalize(seg + 1)

        slot = jax.lax.rem(e, _RING)
        y = slab_ref[pl.ds(slot * _B, _B), :]
        a0 = coef_ref[pl.ds(2 * seg, 1), :]
        c0 = coef_ref[pl.ds(2 * seg + 1, 1), :]

        @pl.when(jnp.logical_not(straddle))
        def _():
            o_ref[...] = y * a0 + c0

        @pl.when(straddle)
        def _():
            rowid = jax.lax.broadcasted_iota(jnp.int32, (_B, 256), 0)
            a1 = coef_ref[pl.ds(2 * seg + 2, 1), :]
            c1 = coef_ref[pl.ds(2 * seg + 3, 1), :]
            o_ref[...] = jnp.where(rowid < split, y * a0 + c0, y * a1 + c1)


def kernel(h, weight, bias, mean_scale):
    n_rows, d = h.shape
    n_blk = n_rows // _B
    hf = h.astype(jnp.float32)
    w2 = weight.astype(jnp.float32).reshape(1, d)
    b2 = bias.astype(jnp.float32).reshape(1, d)
    ms2 = mean_scale.astype(jnp.float32).reshape(1, d)

    out = pl.pallas_call(
        _gn_kernel,
        grid=(n_blk + _LAG,),
        in_specs=[
            pl.BlockSpec((_B, d), lambda s: (jnp.minimum(s, n_blk - 1), 0)),
            pl.BlockSpec((1, d), lambda s: (0, 0)),
            pl.BlockSpec((1, d), lambda s: (0, 0)),
            pl.BlockSpec((1, d), lambda s: (0, 0)),
        ],
        out_specs=pl.BlockSpec(
            (_B, d), lambda s: (jnp.maximum(s - _LAG, 0), 0)
        ),
        out_shape=jax.ShapeDtypeStruct((n_rows, d), jnp.float32),
        scratch_shapes=[
            pltpu.VMEM((_RING * _B, 256), jnp.float32),
            pltpu.VMEM((16, 256), jnp.float32),
            pltpu.VMEM((16, 256), jnp.float32),
        ],
    )(hf, w2, b2, ms2)

    return out.astype(h.dtype)


# speedup vs baseline: 5.5989x; 1.0152x over previous
import jax
import jax.numpy as jnp
from jax.experimental import pallas as pl
from jax.experimental.pallas import tpu as pltpu

_GROUP = 12500
_B = 5000
_LAG = 3
_RING = 4
_EPS = 1e-6


def _gn_kernel(h_ref, w_ref, b_ref, ms_ref, o_ref, slab_ref, sums_ref, coef_ref):
    s = pl.program_id(0)
    n_in = pl.num_programs(0) - _LAG
    inv_n = 1.0 / _GROUP

    @pl.when(s < n_in)
    def _ingest():
        x = h_ref[...]
        slot = jax.lax.rem(s, _RING)
        slab_ref[pl.ds(slot * _B, _B), :] = x
        pos = jax.lax.rem(s * _B, _GROUP)
        seg = jax.lax.div(s * _B, _GROUP)
        split = _GROUP - pos

        def psums(xm):
            return (jnp.sum(xm, axis=0, keepdims=True),
                    jnp.sum(xm * xm, axis=0, keepdims=True))

        @pl.when(pos == 0)
        def _():
            ps, pss = psums(x)
            sums_ref[pl.ds(2 * seg, 1), :] = ps
            sums_ref[pl.ds(2 * seg + 1, 1), :] = pss

        @pl.when((pos > 0) & (pos + _B <= _GROUP))
        def _():
            ps, pss = psums(x)
            sums_ref[pl.ds(2 * seg, 1), :] += ps
            sums_ref[pl.ds(2 * seg + 1, 1), :] += pss

        @pl.when(pos + _B > _GROUP)
        def _():
            rowid = jax.lax.broadcasted_iota(jnp.int32, (_B, 256), 0)
            lo = rowid < split
            ps, pss = psums(jnp.where(lo, x, 0.0))
            sums_ref[pl.ds(2 * seg, 1), :] += ps
            sums_ref[pl.ds(2 * seg + 1, 1), :] += pss
            ps2, pss2 = psums(jnp.where(lo, 0.0, x))
            sums_ref[pl.ds(2 * seg + 2, 1), :] = ps2
            sums_ref[pl.ds(2 * seg + 3, 1), :] = pss2

    @pl.when(s >= _LAG)
    def _emit():
        e = s - _LAG
        pos = jax.lax.rem(e * _B, _GROUP)
        seg = jax.lax.div(e * _B, _GROUP)
        split = _GROUP - pos
        straddle = pos + _B > _GROUP

        def finalize(j):
            sm = sums_ref[pl.ds(2 * j, 1), :]
            ss = sums_ref[pl.ds(2 * j + 1, 1), :]
            m = sm * inv_n
            mm = m * ms_ref[...]
            var = ss * inv_n - (2.0 * m - mm) * mm
            a = w_ref[...] * jax.lax.rsqrt(var + _EPS)
            coef_ref[pl.ds(2 * j, 1), :] = a
            coef_ref[pl.ds(2 * j + 1, 1), :] = b_ref[...] - a * mm

        @pl.when(pos == 0)
        def _():
            finalize(seg)

        @pl.when(straddle)
        def _():
            finalize(seg + 1)

        slot = jax.lax.rem(e, _RING)
        y = slab_ref[pl.ds(slot * _B, _B), :]
        a0 = coef_ref[pl.ds(2 * seg, 1), :]
        c0 = coef_ref[pl.ds(2 * seg + 1, 1), :]

        @pl.when(jnp.logical_not(straddle))
        def _():
            o_ref[...] = y * a0 + c0

        @pl.when(straddle)
        def _():
            rowid = jax.lax.broadcasted_iota(jnp.int32, (_B, 256), 0)
            a1 = coef_ref[pl.ds(2 * seg + 2, 1), :]
            c1 = coef_ref[pl.ds(2 * seg + 3, 1), :]
            o_ref[...] = jnp.where(rowid < split, y * a0 + c0, y * a1 + c1)


def kernel(h, weight, bias, mean_scale):
    n_rows, d = h.shape
    n_blk = n_rows // _B
    hf = h.astype(jnp.float32)
    w2 = weight.astype(jnp.float32).reshape(1, d)
    b2 = bias.astype(jnp.float32).reshape(1, d)
    ms2 = mean_scale.astype(jnp.float32).reshape(1, d)

    out = pl.pallas_call(
        _gn_kernel,
        grid=(n_blk + _LAG,),
        in_specs=[
            pl.BlockSpec((_B, d), lambda s: (jnp.minimum(s, n_blk - 1), 0)),
            pl.BlockSpec((1, d), lambda s: (0, 0)),
            pl.BlockSpec((1, d), lambda s: (0, 0)),
            pl.BlockSpec((1, d), lambda s: (0, 0)),
        ],
        out_specs=pl.BlockSpec(
            (_B, d), lambda s: (jnp.maximum(s - _LAG, 0), 0)
        ),
        out_shape=jax.ShapeDtypeStruct((n_rows, d), jnp.float32),
        scratch_shapes=[
            pltpu.VMEM((_RING * _B, 256), jnp.float32),
            pltpu.VMEM((16, 256), jnp.float32),
            pltpu.VMEM((16, 256), jnp.float32),
        ],
    )(hf, w2, b2, ms2)

    return out.astype(h.dtype)
